# Initial kernel scaffold; baseline (speedup 1.0000x reference)
import jax
import jax.numpy as jnp
from jax.experimental import pallas as pl


def kernel(t, x, HG_poi_src, HG_poi_tar, e):
    raise NotImplementedError("write your pallas kernel here")


# speedup vs baseline: 3.5695x; 3.5695x over previous
import jax
import jax.numpy as jnp
from jax.experimental import pallas as pl

N = 10000
K = 256
D = 128
BN = 2000
BK = 32


def _tmp_body(tar_ref, x_ref, tmp_ref):
    tmp_ref[...] = jnp.dot(
        tar_ref[...], x_ref[...], preferred_element_type=jnp.float32
    )


def _out_body(src_ref, tmp_ref, x_ref, e_ref, out_ref):
    out_ref[...] = (
        jnp.dot(src_ref[...], tmp_ref[...], preferred_element_type=jnp.float32)
        + e_ref[...]
        - x_ref[...]
    )


def kernel(t, x, HG_poi_src, HG_poi_tar, e):
    del t
    tmp = pl.pallas_call(
        _tmp_body,
        grid=(K // BK,),
        in_specs=[
            pl.BlockSpec((BK, N), lambda i: (i, 0)),
            pl.BlockSpec((N, D), lambda i: (0, 0)),
        ],
        out_specs=pl.BlockSpec((BK, D), lambda i: (i, 0)),
        out_shape=jax.ShapeDtypeStruct((K, D), jnp.float32),
    )(HG_poi_tar, x)

    f = pl.pallas_call(
        _out_body,
        grid=(N // BN,),
        in_specs=[
            pl.BlockSpec((BN, K), lambda i: (i, 0)),
            pl.BlockSpec((K, D), lambda i: (0, 0)),
            pl.BlockSpec((BN, D), lambda i: (i, 0)),
            pl.BlockSpec((BN, D), lambda i: (i, 0)),
        ],
        out_specs=pl.BlockSpec((BN, D), lambda i: (i, 0)),
        out_shape=jax.ShapeDtypeStruct((N, D), jnp.float32),
    )(HG_poi_src, tmp, x, e)
    return f
